# baseline (speedup 1.0000x reference)
import functools

import jax
import jax.numpy as jnp
from jax import lax
from jax.experimental import pallas as pl
from jax.experimental.pallas import tpu as pltpu
from jax.experimental.pallas import tpu_sc as plsc

F = 256
N_USERS = 5000
N_ITEMS = 5000
N_EDGES = 160000
NC, NS = 2, 16
NW = NC * NS
E_PAD = 163840
N_PAD = 5120


def _pick_chunk(n):
    for c in (128, 120, 112, 104, 96, 88, 80, 72, 64, 56, 48, 40, 32, 24, 16, 8):
        if n % c == 0:
            return c
    raise ValueError(n)


def _sc_gather_rows(table, idx):
    B = idx.shape[0]
    V, D = table.shape
    b_per_w = B // NW
    C = _pick_chunk(b_per_w)
    n = b_per_w // C
    assert n % 2 == 0
    mesh = plsc.VectorSubcoreMesh(core_axis_name="c", subcore_axis_name="s")

    @functools.partial(
        pl.kernel,
        mesh=mesh,
        out_type=jax.ShapeDtypeStruct((B, D), jnp.float32),
        scratch_types=[
            pltpu.VMEM((C,), jnp.int32),
            pltpu.VMEM((C,), jnp.int32),
            pltpu.VMEM((C, D), jnp.float32),
            pltpu.VMEM((C, D), jnp.float32),
            pltpu.SemaphoreType.DMA,
            pltpu.SemaphoreType.DMA,
            pltpu.SemaphoreType.DMA,
            pltpu.SemaphoreType.DMA,
            pltpu.SemaphoreType.DMA,
            pltpu.SemaphoreType.DMA,
        ],
    )
    def gather_k(table_hbm, idx_hbm, out_hbm, idx0, idx1, rows0, rows1,
                 is0, is1, gs0, gs1, os0, os1):
        wid = lax.axis_index("s") * NC + lax.axis_index("c")
        w0 = wid * b_per_w
        idxs = (idx0, idx1)
        rows = (rows0, rows1)
        isem = (is0, is1)
        gsem = (gs0, gs1)
        osem = (os0, os1)

        def start_idx(j, s):
            pltpu.async_copy(idx_hbm.at[pl.ds(w0 + j * C, C)], idxs[s],
                             isem[s])

        def wait_idx(s):
            pltpu.make_async_copy(idx_hbm.at[pl.ds(w0, C)], idxs[s],
                                  isem[s]).wait()

        def start_gather(s):
            pltpu.async_copy(table_hbm.at[idxs[s]], rows[s], gsem[s])

        def wait_gather(s):
            pltpu.make_async_copy(out_hbm.at[pl.ds(w0, C)], rows[s],
                                  gsem[s]).wait()

        def start_out(j, s):
            pltpu.async_copy(rows[s], out_hbm.at[pl.ds(w0 + j * C, C)],
                             osem[s])

        def wait_out(s):
            pltpu.make_async_copy(rows[s], out_hbm.at[pl.ds(w0, C)],
                                  osem[s]).wait()

        start_idx(0, 0)
        start_idx(1, 1)
        wait_idx(0)
        start_gather(0)

        def process(j, s):
            @pl.when(j + 1 < n)
            def _():
                wait_idx(1 - s)

                @pl.when(j >= 1)
                def _():
                    wait_out(1 - s)

                start_gather(1 - s)

            wait_gather(s)
            start_out(j, s)

            @pl.when(j + 2 < n)
            def _():
                start_idx(j + 2, s)

        def body(g, carry):
            process(2 * g, 0)
            process(2 * g + 1, 1)
            return carry

        lax.fori_loop(0, n // 2, body, 0)
        wait_out((n - 2) % 2)
        wait_out((n - 1) % 2)

    return gather_k(table, idx)


def _sc_segment_sum(m_items, m_users, dst_idx, src_idx):
    C = 512
    n_iter = E_PAD // C
    mesh = plsc.VectorSubcoreMesh(core_axis_name="c", subcore_axis_name="s")

    @functools.partial(
        pl.kernel,
        mesh=mesh,
        out_type=(
            jax.ShapeDtypeStruct((F * N_PAD,), jnp.float32),
            jax.ShapeDtypeStruct((F * N_PAD,), jnp.float32),
        ),
        scratch_types=[
            pltpu.VMEM((C,), jnp.int32),
            pltpu.VMEM((C,), jnp.int32),
            pltpu.VMEM((16, C), jnp.float32),
            pltpu.VMEM((16, C), jnp.float32),
            pltpu.VMEM((C * 16,), jnp.float32),
            pltpu.VMEM((16 * N_PAD,), jnp.float32),
            pltpu.SemaphoreType.DMA,
            pltpu.SemaphoreType.DMA,
        ],
        compiler_params=pltpu.CompilerParams(needs_layout_passes=False),
    )
    def seg_k(mi_hbm, mu_hbm, di_hbm, si_hbm,
              item_out, user_out, idx0, idx1, rows0, rows1, flat_v, acc,
              sem0, sem1):
        cid = lax.axis_index("c")
        sid = lax.axis_index("s")
        col0 = sid * 16
        iot = lax.iota(jnp.int32, 16)
        iot16 = iot * 16
        iotN = iot * N_PAD
        zv = jnp.zeros((16,), jnp.float32)
        idxs = (idx0, idx1)
        rows = (rows0, rows1)
        sems = (sem0, sem1)

        def run(m_hbm, i_hbm, out_hbm):
            def start(j, s):
                base = j * C
                pltpu.async_copy(i_hbm.at[pl.ds(base, C)], idxs[s], sems[s])
                pltpu.async_copy(
                    m_hbm.at[pl.ds(col0, 16), pl.ds(base, C)], rows[s],
                    sems[s])

            def wait(s):
                pltpu.make_async_copy(i_hbm.at[pl.ds(0, C)], idxs[s],
                                      sems[s]).wait()
                pltpu.make_async_copy(
                    m_hbm.at[pl.ds(col0, 16), pl.ds(0, C)], rows[s],
                    sems[s]).wait()

            def zbody(r, carry):
                for u in range(8):
                    acc[pl.ds(r * 128 + u * 16, 16)] = zv
                return carry

            start(0, 0)
            start(1, 1)
            lax.fori_loop(0, N_PAD // 8, zbody, 0)

            def process(j, s):
                wait(s)

                def tbody(b, carry):
                    boff = jnp.full((16,), b * 256, jnp.int32)
                    for c in range(16):
                        v = rows[s][c, pl.ds(b * 16, 16)]
                        plsc.store_scatter(flat_v, [(iot16 + c) + boff], v)
                    return carry

                lax.fori_loop(0, C // 16, tbody, 0)

                @pl.when(j + 2 < n_iter)
                def _():
                    pltpu.async_copy(
                        m_hbm.at[pl.ds(col0, 16), pl.ds((j + 2) * C, C)],
                        rows[s], sems[s])

                def ebody(g, c2):
                    g256 = g * 256
                    dvec = idxs[s][pl.ds(g * 16, 16)]
                    for l in range(16):
                        bv = jnp.take_along_axis(
                            dvec, jnp.full((16,), l, jnp.int32), axis=0,
                            mode="promise_in_bounds")
                        v = flat_v[pl.ds(g256 + l * 16, 16)]
                        plsc.addupdate_scatter(acc, [iotN + bv], v)
                    return c2

                lax.fori_loop(0, C // 16, ebody, 0)

                @pl.when(j + 2 < n_iter)
                def _():
                    pltpu.async_copy(i_hbm.at[pl.ds((j + 2) * C, C)],
                                     idxs[s], sems[s])

            def body(g, carry):
                process(2 * g, 0)
                process(2 * g + 1, 1)
                return carry

            lax.fori_loop(0, n_iter // 2, body, 0)
            pltpu.sync_copy(acc, out_hbm.at[pl.ds(col0 * N_PAD, 16 * N_PAD)])

        @pl.when(cid == 0)
        def _():
            run(mi_hbm, di_hbm, item_out)

        @pl.when(cid == 1)
        def _():
            run(mu_hbm, si_hbm, user_out)

    return seg_k(m_items, m_users, dst_idx, src_idx)


_BLK = 256
_NBLK = E_PAD // _BLK


def _edge_dot_kernel(hs_ref, hd_ref, e_ref):
    i = pl.program_id(0)
    prod = hs_ref[...] * hd_ref[...]
    s = jnp.sum(prod, axis=1, keepdims=True) * (1.0 / 16.0)
    row = i * _BLK + lax.broadcasted_iota(jnp.int32, (_BLK, 1), 0)
    s = jnp.where(row < N_EDGES, s, -1e30)
    e_ref[...] = s.reshape(1, 1, _BLK)


def _tc_edge_dots(hs_e, hd_e):
    return pl.pallas_call(
        _edge_dot_kernel,
        grid=(_NBLK,),
        in_specs=[
            pl.BlockSpec((_BLK, F), lambda i: (i, 0)),
            pl.BlockSpec((_BLK, F), lambda i: (i, 0)),
        ],
        out_specs=pl.BlockSpec((1, 1, _BLK), lambda i: (i, 0, 0)),
        out_shape=jax.ShapeDtypeStruct((_NBLK, 1, _BLK), jnp.float32),
    )(hs_e, hd_e)


def _softmax_kernel(e_ref, a_ref):
    e = e_ref[...]
    m = jnp.max(e)
    ex = jnp.exp(e - m)
    a_ref[...] = ex * (1.0 / jnp.sum(ex))


def _tc_softmax(e3):
    return pl.pallas_call(
        _softmax_kernel,
        out_shape=jax.ShapeDtypeStruct((_NBLK, 1, _BLK), jnp.float32),
    )(e3)


def _msg_kernel(hs_ref, hd_ref, a_ref, ws_ref, bs_ref, wd_ref, bd_ref,
                mi_ref, mu_ref):
    alpha = a_ref[...].reshape(1, _BLK)
    fsT = jnp.maximum(
        lax.dot_general(ws_ref[...], hs_ref[...], (((1,), (1,)), ((), ())),
                        precision=lax.Precision.HIGHEST,
                        preferred_element_type=jnp.float32) + bs_ref[...], 0.0)
    mi_ref[...] = fsT * alpha
    fdT = jnp.maximum(
        lax.dot_general(wd_ref[...], hd_ref[...], (((1,), (1,)), ((), ())),
                        precision=lax.Precision.HIGHEST,
                        preferred_element_type=jnp.float32) + bd_ref[...], 0.0)
    mu_ref[...] = fdT * alpha


def _tc_messages(hs_e, hd_e, alpha3, Ws, bs2, Wd, bd2):
    return pl.pallas_call(
        _msg_kernel,
        grid=(_NBLK,),
        in_specs=[
            pl.BlockSpec((_BLK, F), lambda i: (i, 0)),
            pl.BlockSpec((_BLK, F), lambda i: (i, 0)),
            pl.BlockSpec((1, 1, _BLK), lambda i: (i, 0, 0)),
            pl.BlockSpec((F, F), lambda i: (0, 0)),
            pl.BlockSpec((F, 1), lambda i: (0, 0)),
            pl.BlockSpec((F, F), lambda i: (0, 0)),
            pl.BlockSpec((F, 1), lambda i: (0, 0)),
        ],
        out_specs=[
            pl.BlockSpec((F, _BLK), lambda i: (0, i)),
            pl.BlockSpec((F, _BLK), lambda i: (0, i)),
        ],
        out_shape=[
            jax.ShapeDtypeStruct((F, E_PAD), jnp.float32),
            jax.ShapeDtypeStruct((F, E_PAD), jnp.float32),
        ],
    )(hs_e, hd_e, alpha3, Ws, bs2, Wd, bd2)


def kernel(feat, user_ids, item_ids, edge_src, edge_dst,
           W_src, b_src, W_dst, b_dst):
    uid_p = jnp.pad(user_ids.astype(jnp.int32), (0, N_PAD - N_USERS))
    iid_p = jnp.pad(item_ids.astype(jnp.int32), (0, N_PAD - N_ITEMS))
    es_p = jnp.pad(edge_src.astype(jnp.int32), (0, E_PAD - N_EDGES))
    ed_p = jnp.pad(edge_dst.astype(jnp.int32), (0, E_PAD - N_EDGES))

    h_src = _sc_gather_rows(feat, uid_p)
    h_dst = _sc_gather_rows(feat, iid_p)

    hs_e = _sc_gather_rows(h_src, es_p)
    hd_e = _sc_gather_rows(h_dst, ed_p)

    e3 = _tc_edge_dots(hs_e, hd_e)
    alpha3 = _tc_softmax(e3)

    m_items_T, m_users_T = _tc_messages(
        hs_e, hd_e, alpha3,
        W_src, b_src.reshape(F, 1),
        W_dst, b_dst.reshape(F, 1))

    item_1d, user_1d = _sc_segment_sum(m_items_T, m_users_T, ed_p, es_p)
    item_T = item_1d.reshape(F, N_PAD)
    user_T = user_1d.reshape(F, N_PAD)

    return jnp.concatenate(
        [user_T[:, :N_USERS].T, item_T[:, :N_ITEMS].T], axis=0)


# speedup vs baseline: 1.4060x; 1.4060x over previous
import functools

import jax
import jax.numpy as jnp
from jax import lax
from jax.experimental import pallas as pl
from jax.experimental.pallas import tpu as pltpu
from jax.experimental.pallas import tpu_sc as plsc

F = 256
N_USERS = 5000
N_ITEMS = 5000
N_EDGES = 160000
NC, NS = 2, 16
NW = NC * NS
E_PAD = 163840
N_PAD = 5120


def _pick_chunk(n):
    for c in (128, 120, 112, 104, 96, 88, 80, 72, 64, 56, 48, 40, 32, 24, 16, 8):
        if n % c == 0:
            return c
    raise ValueError(n)


def _sc_gather_rows(table, idx):
    B = idx.shape[0]
    V, D = table.shape
    b_per_w = B // NW
    C = _pick_chunk(b_per_w)
    n = b_per_w // C
    assert n % 2 == 0
    mesh = plsc.VectorSubcoreMesh(core_axis_name="c", subcore_axis_name="s")

    @functools.partial(
        pl.kernel,
        mesh=mesh,
        out_type=jax.ShapeDtypeStruct((B, D), jnp.float32),
        scratch_types=[
            pltpu.VMEM((C,), jnp.int32),
            pltpu.VMEM((C,), jnp.int32),
            pltpu.VMEM((C, D), jnp.float32),
            pltpu.VMEM((C, D), jnp.float32),
            pltpu.SemaphoreType.DMA,
            pltpu.SemaphoreType.DMA,
            pltpu.SemaphoreType.DMA,
            pltpu.SemaphoreType.DMA,
            pltpu.SemaphoreType.DMA,
            pltpu.SemaphoreType.DMA,
        ],
    )
    def gather_k(table_hbm, idx_hbm, out_hbm, idx0, idx1, rows0, rows1,
                 is0, is1, gs0, gs1, os0, os1):
        wid = lax.axis_index("s") * NC + lax.axis_index("c")
        w0 = wid * b_per_w
        idxs = (idx0, idx1)
        rows = (rows0, rows1)
        isem = (is0, is1)
        gsem = (gs0, gs1)
        osem = (os0, os1)

        def start_idx(j, s):
            pltpu.async_copy(idx_hbm.at[pl.ds(w0 + j * C, C)], idxs[s],
                             isem[s])

        def wait_idx(s):
            pltpu.make_async_copy(idx_hbm.at[pl.ds(w0, C)], idxs[s],
                                  isem[s]).wait()

        def start_gather(s):
            pltpu.async_copy(table_hbm.at[idxs[s]], rows[s], gsem[s])

        def wait_gather(s):
            pltpu.make_async_copy(out_hbm.at[pl.ds(w0, C)], rows[s],
                                  gsem[s]).wait()

        def start_out(j, s):
            pltpu.async_copy(rows[s], out_hbm.at[pl.ds(w0 + j * C, C)],
                             osem[s])

        def wait_out(s):
            pltpu.make_async_copy(rows[s], out_hbm.at[pl.ds(w0, C)],
                                  osem[s]).wait()

        start_idx(0, 0)
        start_idx(1, 1)
        wait_idx(0)
        start_gather(0)

        def process(j, s):
            @pl.when(j + 1 < n)
            def _():
                wait_idx(1 - s)

                @pl.when(j >= 1)
                def _():
                    wait_out(1 - s)

                start_gather(1 - s)

            wait_gather(s)
            start_out(j, s)

            @pl.when(j + 2 < n)
            def _():
                start_idx(j + 2, s)

        def body(g, carry):
            process(2 * g, 0)
            process(2 * g + 1, 1)
            return carry

        lax.fori_loop(0, n // 2, body, 0)
        wait_out((n - 2) % 2)
        wait_out((n - 1) % 2)

    return gather_k(table, idx)


def _sc_segment_sum(m_items, m_users, dst_idx, src_idx):
    C = 512
    n_iter = E_PAD // C
    mesh = plsc.VectorSubcoreMesh(core_axis_name="c", subcore_axis_name="s")

    @functools.partial(
        pl.kernel,
        mesh=mesh,
        out_type=(
            jax.ShapeDtypeStruct((F * N_PAD,), jnp.float32),
            jax.ShapeDtypeStruct((F * N_PAD,), jnp.float32),
        ),
        scratch_types=[
            pltpu.VMEM((C,), jnp.int32),
            pltpu.VMEM((C,), jnp.int32),
            pltpu.VMEM((16, C), jnp.float32),
            pltpu.VMEM((16, C), jnp.float32),
            pltpu.VMEM((C * 16,), jnp.float32),
            pltpu.VMEM((16 * N_PAD,), jnp.float32),
            pltpu.SemaphoreType.DMA,
            pltpu.SemaphoreType.DMA,
        ],
        compiler_params=pltpu.CompilerParams(needs_layout_passes=False),
    )
    def seg_k(mi_hbm, mu_hbm, di_hbm, si_hbm,
              item_out, user_out, idx0, idx1, rows0, rows1, flat_v, acc,
              sem0, sem1):
        cid = lax.axis_index("c")
        sid = lax.axis_index("s")
        col0 = sid * 16
        iot = lax.iota(jnp.int32, 16)
        iot16 = iot * 16
        iotN = iot * N_PAD
        zv = jnp.zeros((16,), jnp.float32)
        idxs = (idx0, idx1)
        rows = (rows0, rows1)
        sems = (sem0, sem1)

        def run(m_hbm, i_hbm, out_hbm):
            def start(j, s):
                base = j * C
                pltpu.async_copy(i_hbm.at[pl.ds(base, C)], idxs[s], sems[s])
                pltpu.async_copy(
                    m_hbm.at[pl.ds(col0, 16), pl.ds(base, C)], rows[s],
                    sems[s])

            def wait(s):
                pltpu.make_async_copy(i_hbm.at[pl.ds(0, C)], idxs[s],
                                      sems[s]).wait()
                pltpu.make_async_copy(
                    m_hbm.at[pl.ds(col0, 16), pl.ds(0, C)], rows[s],
                    sems[s]).wait()

            def zbody(r, carry):
                for u in range(8):
                    acc[pl.ds(r * 128 + u * 16, 16)] = zv
                return carry

            start(0, 0)
            start(1, 1)
            lax.fori_loop(0, N_PAD // 8, zbody, 0)

            def process(j, s):
                wait(s)

                def tbody(b, carry):
                    boff = jnp.full((16,), b * 256, jnp.int32)
                    for c in range(16):
                        v = rows[s][c, pl.ds(b * 16, 16)]
                        skew = iot16 + ((iot + c) & 15)
                        plsc.store_scatter(flat_v, [skew + boff], v)
                    return carry

                lax.fori_loop(0, C // 16, tbody, 0)

                @pl.when(j + 2 < n_iter)
                def _():
                    pltpu.async_copy(
                        m_hbm.at[pl.ds(col0, 16), pl.ds((j + 2) * C, C)],
                        rows[s], sems[s])

                def ebody(g, c2):
                    g256 = g * 256
                    dvec16 = idxs[s][pl.ds(g * 16, 16)] * 16
                    for l in range(16):
                        bv16 = jnp.take_along_axis(
                            dvec16, jnp.full((16,), l, jnp.int32), axis=0,
                            mode="promise_in_bounds")
                        v = flat_v[pl.ds(g256 + l * 16, 16)]
                        plsc.addupdate_scatter(
                            acc, [bv16 + ((iot - l) & 15)], v)
                    return c2

                lax.fori_loop(0, C // 16, ebody, 0)

                @pl.when(j + 2 < n_iter)
                def _():
                    pltpu.async_copy(i_hbm.at[pl.ds((j + 2) * C, C)],
                                     idxs[s], sems[s])

            def body(g, carry):
                process(2 * g, 0)
                process(2 * g + 1, 1)
                return carry

            lax.fori_loop(0, n_iter // 2, body, 0)
            pltpu.sync_copy(acc, out_hbm.at[pl.ds(col0 * N_PAD, 16 * N_PAD)])

        @pl.when(cid == 0)
        def _():
            run(mi_hbm, di_hbm, item_out)

        @pl.when(cid == 1)
        def _():
            run(mu_hbm, si_hbm, user_out)

    return seg_k(m_items, m_users, dst_idx, src_idx)


_BLK = 256
_NBLK = E_PAD // _BLK


def _edge_dot_kernel(hs_ref, hd_ref, e_ref):
    i = pl.program_id(0)
    prod = hs_ref[...] * hd_ref[...]
    s = jnp.sum(prod, axis=1, keepdims=True) * (1.0 / 16.0)
    row = i * _BLK + lax.broadcasted_iota(jnp.int32, (_BLK, 1), 0)
    s = jnp.where(row < N_EDGES, s, -1e30)
    e_ref[...] = s.reshape(1, 1, _BLK)


def _tc_edge_dots(hs_e, hd_e):
    return pl.pallas_call(
        _edge_dot_kernel,
        grid=(_NBLK,),
        in_specs=[
            pl.BlockSpec((_BLK, F), lambda i: (i, 0)),
            pl.BlockSpec((_BLK, F), lambda i: (i, 0)),
        ],
        out_specs=pl.BlockSpec((1, 1, _BLK), lambda i: (i, 0, 0)),
        out_shape=jax.ShapeDtypeStruct((_NBLK, 1, _BLK), jnp.float32),
    )(hs_e, hd_e)


def _softmax_kernel(e_ref, a_ref):
    e = e_ref[...]
    m = jnp.max(e)
    ex = jnp.exp(e - m)
    a_ref[...] = ex * (1.0 / jnp.sum(ex))


def _tc_softmax(e3):
    return pl.pallas_call(
        _softmax_kernel,
        out_shape=jax.ShapeDtypeStruct((_NBLK, 1, _BLK), jnp.float32),
    )(e3)


def _msg_kernel(hs_ref, hd_ref, a_ref, ws_ref, bs_ref, wd_ref, bd_ref,
                mi_ref, mu_ref):
    alpha = a_ref[...].reshape(1, _BLK)
    fsT = jnp.maximum(
        lax.dot_general(ws_ref[...], hs_ref[...], (((1,), (1,)), ((), ())),
                        precision=lax.Precision.HIGHEST,
                        preferred_element_type=jnp.float32) + bs_ref[...], 0.0)
    mi_ref[...] = fsT * alpha
    fdT = jnp.maximum(
        lax.dot_general(wd_ref[...], hd_ref[...], (((1,), (1,)), ((), ())),
                        precision=lax.Precision.HIGHEST,
                        preferred_element_type=jnp.float32) + bd_ref[...], 0.0)
    mu_ref[...] = fdT * alpha


def _tc_messages(hs_e, hd_e, alpha3, Ws, bs2, Wd, bd2):
    return pl.pallas_call(
        _msg_kernel,
        grid=(_NBLK,),
        in_specs=[
            pl.BlockSpec((_BLK, F), lambda i: (i, 0)),
            pl.BlockSpec((_BLK, F), lambda i: (i, 0)),
            pl.BlockSpec((1, 1, _BLK), lambda i: (i, 0, 0)),
            pl.BlockSpec((F, F), lambda i: (0, 0)),
            pl.BlockSpec((F, 1), lambda i: (0, 0)),
            pl.BlockSpec((F, F), lambda i: (0, 0)),
            pl.BlockSpec((F, 1), lambda i: (0, 0)),
        ],
        out_specs=[
            pl.BlockSpec((F, _BLK), lambda i: (0, i)),
            pl.BlockSpec((F, _BLK), lambda i: (0, i)),
        ],
        out_shape=[
            jax.ShapeDtypeStruct((F, E_PAD), jnp.float32),
            jax.ShapeDtypeStruct((F, E_PAD), jnp.float32),
        ],
    )(hs_e, hd_e, alpha3, Ws, bs2, Wd, bd2)


def kernel(feat, user_ids, item_ids, edge_src, edge_dst,
           W_src, b_src, W_dst, b_dst):
    uid_p = jnp.pad(user_ids.astype(jnp.int32), (0, N_PAD - N_USERS))
    iid_p = jnp.pad(item_ids.astype(jnp.int32), (0, N_PAD - N_ITEMS))
    es_p = jnp.pad(edge_src.astype(jnp.int32), (0, E_PAD - N_EDGES))
    ed_p = jnp.pad(edge_dst.astype(jnp.int32), (0, E_PAD - N_EDGES))

    h_src = _sc_gather_rows(feat, uid_p)
    h_dst = _sc_gather_rows(feat, iid_p)

    hs_e = _sc_gather_rows(h_src, es_p)
    hd_e = _sc_gather_rows(h_dst, ed_p)

    e3 = _tc_edge_dots(hs_e, hd_e)
    alpha3 = _tc_softmax(e3)

    m_items_T, m_users_T = _tc_messages(
        hs_e, hd_e, alpha3,
        W_src, b_src.reshape(F, 1),
        W_dst, b_dst.reshape(F, 1))

    item_1d, user_1d = _sc_segment_sum(m_items_T, m_users_T, ed_p, es_p)
    item_new = item_1d.reshape(16, N_PAD, 16).transpose(1, 0, 2).reshape(
        N_PAD, F)
    user_new = user_1d.reshape(16, N_PAD, 16).transpose(1, 0, 2).reshape(
        N_PAD, F)

    return jnp.concatenate(
        [user_new[:N_USERS], item_new[:N_ITEMS]], axis=0)


# speedup vs baseline: 1.7410x; 1.2383x over previous
import functools

import jax
import jax.numpy as jnp
from jax import lax
from jax.experimental import pallas as pl
from jax.experimental.pallas import tpu as pltpu
from jax.experimental.pallas import tpu_sc as plsc

F = 256
N_USERS = 5000
N_ITEMS = 5000
N_EDGES = 160000
NC, NS = 2, 16
NW = NC * NS
E_PAD = 163840
N_PAD = 5120


def _pick_chunk(n):
    for c in (128, 120, 112, 104, 96, 88, 80, 72, 64, 56, 48, 40, 32, 24, 16, 8):
        if n % c == 0:
            return c
    raise ValueError(n)


def _sc_gather_rows(table, idx):
    B = idx.shape[0]
    V, D = table.shape
    b_per_w = B // NW
    C = _pick_chunk(b_per_w)
    n = b_per_w // C
    assert n % 2 == 0
    mesh = plsc.VectorSubcoreMesh(core_axis_name="c", subcore_axis_name="s")

    @functools.partial(
        pl.kernel,
        mesh=mesh,
        out_type=jax.ShapeDtypeStruct((B, D), jnp.float32),
        scratch_types=[
            pltpu.VMEM((C,), jnp.int32),
            pltpu.VMEM((C,), jnp.int32),
            pltpu.VMEM((C, D), jnp.float32),
            pltpu.VMEM((C, D), jnp.float32),
            pltpu.SemaphoreType.DMA,
            pltpu.SemaphoreType.DMA,
            pltpu.SemaphoreType.DMA,
            pltpu.SemaphoreType.DMA,
            pltpu.SemaphoreType.DMA,
            pltpu.SemaphoreType.DMA,
        ],
    )
    def gather_k(table_hbm, idx_hbm, out_hbm, idx0, idx1, rows0, rows1,
                 is0, is1, gs0, gs1, os0, os1):
        wid = lax.axis_index("s") * NC + lax.axis_index("c")
        w0 = wid * b_per_w
        idxs = (idx0, idx1)
        rows = (rows0, rows1)
        isem = (is0, is1)
        gsem = (gs0, gs1)
        osem = (os0, os1)

        def start_idx(j, s):
            pltpu.async_copy(idx_hbm.at[pl.ds(w0 + j * C, C)], idxs[s],
                             isem[s])

        def wait_idx(s):
            pltpu.make_async_copy(idx_hbm.at[pl.ds(w0, C)], idxs[s],
                                  isem[s]).wait()

        def start_gather(s):
            pltpu.async_copy(table_hbm.at[idxs[s]], rows[s], gsem[s])

        def wait_gather(s):
            pltpu.make_async_copy(out_hbm.at[pl.ds(w0, C)], rows[s],
                                  gsem[s]).wait()

        def start_out(j, s):
            pltpu.async_copy(rows[s], out_hbm.at[pl.ds(w0 + j * C, C)],
                             osem[s])

        def wait_out(s):
            pltpu.make_async_copy(rows[s], out_hbm.at[pl.ds(w0, C)],
                                  osem[s]).wait()

        start_idx(0, 0)
        start_idx(1, 1)
        wait_idx(0)
        start_gather(0)

        def process(j, s):
            @pl.when(j + 1 < n)
            def _():
                wait_idx(1 - s)

                @pl.when(j >= 1)
                def _():
                    wait_out(1 - s)

                start_gather(1 - s)

            wait_gather(s)
            start_out(j, s)

            @pl.when(j + 2 < n)
            def _():
                start_idx(j + 2, s)

        def body(g, carry):
            process(2 * g, 0)
            process(2 * g + 1, 1)
            return carry

        lax.fori_loop(0, n // 2, body, 0)
        wait_out((n - 2) % 2)
        wait_out((n - 1) % 2)

    return gather_k(table, idx)


def _sc_segment_sum(m_items, m_users, dst_idx, src_idx):
    C = 512
    n_iter = E_PAD // C
    mesh = plsc.VectorSubcoreMesh(core_axis_name="c", subcore_axis_name="s")

    @functools.partial(
        pl.kernel,
        mesh=mesh,
        out_type=(
            jax.ShapeDtypeStruct((F * N_PAD,), jnp.float32),
            jax.ShapeDtypeStruct((F * N_PAD,), jnp.float32),
        ),
        scratch_types=[
            pltpu.VMEM((C,), jnp.int32),
            pltpu.VMEM((C,), jnp.int32),
            pltpu.VMEM((16, C), jnp.float32),
            pltpu.VMEM((16, C), jnp.float32),
            pltpu.VMEM((C * 16,), jnp.float32),
            pltpu.VMEM((16 * N_PAD,), jnp.float32),
            pltpu.SemaphoreType.DMA,
            pltpu.SemaphoreType.DMA,
        ],
        compiler_params=pltpu.CompilerParams(needs_layout_passes=False),
    )
    def seg_k(mi_hbm, mu_hbm, di_hbm, si_hbm,
              item_out, user_out, idx0, idx1, rows0, rows1, flat_v, acc,
              sem0, sem1):
        cid = lax.axis_index("c")
        sid = lax.axis_index("s")
        col0 = sid * 16
        iot = lax.iota(jnp.int32, 16)
        iot16 = iot * 16
        iotN = iot * N_PAD
        zv = jnp.zeros((16,), jnp.float32)
        idxs = (idx0, idx1)
        rows = (rows0, rows1)
        sems = (sem0, sem1)

        def run(m_hbm, i_hbm, out_hbm):
            def start(j, s):
                base = j * C
                pltpu.async_copy(i_hbm.at[pl.ds(base, C)], idxs[s], sems[s])
                pltpu.async_copy(
                    m_hbm.at[pl.ds(col0, 16), pl.ds(base, C)], rows[s],
                    sems[s])

            def wait(s):
                pltpu.make_async_copy(i_hbm.at[pl.ds(0, C)], idxs[s],
                                      sems[s]).wait()
                pltpu.make_async_copy(
                    m_hbm.at[pl.ds(col0, 16), pl.ds(0, C)], rows[s],
                    sems[s]).wait()

            def zbody(r, carry):
                for u in range(8):
                    acc[pl.ds(r * 128 + u * 16, 16)] = zv
                return carry

            start(0, 0)
            start(1, 1)
            lax.fori_loop(0, N_PAD // 8, zbody, 0)

            def process(j, s):
                wait(s)

                def tbody(b, carry):
                    boff = jnp.full((16,), b * 256, jnp.int32)
                    for c in range(16):
                        v = rows[s][c, pl.ds(b * 16, 16)]
                        skew = iot16 + ((iot + c) & 15)
                        plsc.store_scatter(flat_v, [skew + boff], v)
                    return carry

                lax.fori_loop(0, C // 16, tbody, 0)

                @pl.when(j + 2 < n_iter)
                def _():
                    pltpu.async_copy(
                        m_hbm.at[pl.ds(col0, 16), pl.ds((j + 2) * C, C)],
                        rows[s], sems[s])

                def ebody(g, c2):
                    g256 = g * 256
                    dvec16 = idxs[s][pl.ds(g * 16, 16)] * 16
                    for l in range(16):
                        bv16 = jnp.take_along_axis(
                            dvec16, jnp.full((16,), l, jnp.int32), axis=0,
                            mode="promise_in_bounds")
                        v = flat_v[pl.ds(g256 + l * 16, 16)]
                        plsc.addupdate_scatter(
                            acc, [bv16 + ((iot - l) & 15)], v)
                    return c2

                lax.fori_loop(0, C // 16, ebody, 0)

                @pl.when(j + 2 < n_iter)
                def _():
                    pltpu.async_copy(i_hbm.at[pl.ds((j + 2) * C, C)],
                                     idxs[s], sems[s])

            def body(g, carry):
                process(2 * g, 0)
                process(2 * g + 1, 1)
                return carry

            lax.fori_loop(0, n_iter // 2, body, 0)
            pltpu.sync_copy(acc, out_hbm.at[pl.ds(col0 * N_PAD, 16 * N_PAD)])

        @pl.when(cid == 0)
        def _():
            run(mi_hbm, di_hbm, item_out)

        @pl.when(cid == 1)
        def _():
            run(mu_hbm, si_hbm, user_out)

    return seg_k(m_items, m_users, dst_idx, src_idx)


_BLK = 1024
_NBLK = E_PAD // _BLK


def _edge_dot_kernel(hs_ref, hd_ref, e_ref):
    i = pl.program_id(0)
    prod = hs_ref[...] * hd_ref[...]
    s = jnp.sum(prod.reshape(8, 128, F), axis=2) * (1.0 / 16.0)
    row = (i * _BLK
           + lax.broadcasted_iota(jnp.int32, (8, 128), 0) * 128
           + lax.broadcasted_iota(jnp.int32, (8, 128), 1))
    s = jnp.where(row < N_EDGES, s, -1e30)
    e_ref[...] = s.reshape(1, 8, 128)


def _tc_edge_dots(hs_e, hd_e):
    return pl.pallas_call(
        _edge_dot_kernel,
        grid=(_NBLK,),
        in_specs=[
            pl.BlockSpec((_BLK, F), lambda i: (i, 0)),
            pl.BlockSpec((_BLK, F), lambda i: (i, 0)),
        ],
        out_specs=pl.BlockSpec((1, 8, 128), lambda i: (i, 0, 0)),
        out_shape=jax.ShapeDtypeStruct((_NBLK, 8, 128), jnp.float32),
    )(hs_e, hd_e)


def _softmax_kernel(e_ref, a_ref):
    e = e_ref[...]
    m = jnp.max(e)
    ex = jnp.exp(e - m)
    a_ref[...] = ex * (1.0 / jnp.sum(ex))


def _tc_softmax(e3):
    return pl.pallas_call(
        _softmax_kernel,
        out_shape=jax.ShapeDtypeStruct((_NBLK, 8, 128), jnp.float32),
    )(e3)


def _msg_kernel(hs_ref, hd_ref, a_ref, ws_ref, bs_ref, wd_ref, bd_ref,
                mi_ref, mu_ref):
    alpha = a_ref[...].reshape(1, _BLK)
    fsT = jnp.maximum(
        lax.dot_general(ws_ref[...], hs_ref[...], (((1,), (1,)), ((), ())),
                        precision=lax.Precision.HIGHEST,
                        preferred_element_type=jnp.float32) + bs_ref[...], 0.0)
    mi_ref[...] = fsT * alpha
    fdT = jnp.maximum(
        lax.dot_general(wd_ref[...], hd_ref[...], (((1,), (1,)), ((), ())),
                        precision=lax.Precision.HIGHEST,
                        preferred_element_type=jnp.float32) + bd_ref[...], 0.0)
    mu_ref[...] = fdT * alpha


def _tc_messages(hs_e, hd_e, alpha3, Ws, bs2, Wd, bd2):
    return pl.pallas_call(
        _msg_kernel,
        grid=(_NBLK,),
        in_specs=[
            pl.BlockSpec((_BLK, F), lambda i: (i, 0)),
            pl.BlockSpec((_BLK, F), lambda i: (i, 0)),
            pl.BlockSpec((1, 8, 128), lambda i: (i, 0, 0)),
            pl.BlockSpec((F, F), lambda i: (0, 0)),
            pl.BlockSpec((F, 1), lambda i: (0, 0)),
            pl.BlockSpec((F, F), lambda i: (0, 0)),
            pl.BlockSpec((F, 1), lambda i: (0, 0)),
        ],
        out_specs=[
            pl.BlockSpec((F, _BLK), lambda i: (0, i)),
            pl.BlockSpec((F, _BLK), lambda i: (0, i)),
        ],
        out_shape=[
            jax.ShapeDtypeStruct((F, E_PAD), jnp.float32),
            jax.ShapeDtypeStruct((F, E_PAD), jnp.float32),
        ],
    )(hs_e, hd_e, alpha3, Ws, bs2, Wd, bd2)


def kernel(feat, user_ids, item_ids, edge_src, edge_dst,
           W_src, b_src, W_dst, b_dst):
    uid_p = jnp.pad(user_ids.astype(jnp.int32), (0, N_PAD - N_USERS))
    iid_p = jnp.pad(item_ids.astype(jnp.int32), (0, N_PAD - N_ITEMS))
    es_p = jnp.pad(edge_src.astype(jnp.int32), (0, E_PAD - N_EDGES))
    ed_p = jnp.pad(edge_dst.astype(jnp.int32), (0, E_PAD - N_EDGES))

    h_src = _sc_gather_rows(feat, uid_p)
    h_dst = _sc_gather_rows(feat, iid_p)

    hs_e = _sc_gather_rows(h_src, es_p)
    hd_e = _sc_gather_rows(h_dst, ed_p)

    e3 = _tc_edge_dots(hs_e, hd_e)
    alpha3 = _tc_softmax(e3)

    m_items_T, m_users_T = _tc_messages(
        hs_e, hd_e, alpha3,
        W_src, b_src.reshape(F, 1),
        W_dst, b_dst.reshape(F, 1))

    item_1d, user_1d = _sc_segment_sum(m_items_T, m_users_T, ed_p, es_p)
    item_new = item_1d.reshape(16, N_PAD, 16).transpose(1, 0, 2).reshape(
        N_PAD, F)
    user_new = user_1d.reshape(16, N_PAD, 16).transpose(1, 0, 2).reshape(
        N_PAD, F)

    return jnp.concatenate(
        [user_new[:N_USERS], item_new[:N_ITEMS]], axis=0)


# speedup vs baseline: 1.8482x; 1.0616x over previous
import functools

import jax
import jax.numpy as jnp
from jax import lax
from jax.experimental import pallas as pl
from jax.experimental.pallas import tpu as pltpu
from jax.experimental.pallas import tpu_sc as plsc

F = 256
N_USERS = 5000
N_ITEMS = 5000
N_EDGES = 160000
NC, NS = 2, 16
NW = NC * NS
E_PAD = 163840
N_PAD = 5120


def _pick_chunk(n):
    for c in (128, 120, 112, 104, 96, 88, 80, 72, 64, 56, 48, 40, 32, 24, 16, 8):
        if n % c == 0:
            return c
    raise ValueError(n)


def _sc_gather_rows(table, idx):
    B = idx.shape[0]
    V, D = table.shape
    b_per_w = B // NW
    C = _pick_chunk(b_per_w)
    n = b_per_w // C
    assert n % 2 == 0
    mesh = plsc.VectorSubcoreMesh(core_axis_name="c", subcore_axis_name="s")

    @functools.partial(
        pl.kernel,
        mesh=mesh,
        out_type=jax.ShapeDtypeStruct((B, D), jnp.float32),
        scratch_types=[
            pltpu.VMEM((C,), jnp.int32),
            pltpu.VMEM((C,), jnp.int32),
            pltpu.VMEM((C, D), jnp.float32),
            pltpu.VMEM((C, D), jnp.float32),
            pltpu.SemaphoreType.DMA,
            pltpu.SemaphoreType.DMA,
            pltpu.SemaphoreType.DMA,
            pltpu.SemaphoreType.DMA,
            pltpu.SemaphoreType.DMA,
            pltpu.SemaphoreType.DMA,
        ],
    )
    def gather_k(table_hbm, idx_hbm, out_hbm, idx0, idx1, rows0, rows1,
                 is0, is1, gs0, gs1, os0, os1):
        wid = lax.axis_index("s") * NC + lax.axis_index("c")
        w0 = wid * b_per_w
        idxs = (idx0, idx1)
        rows = (rows0, rows1)
        isem = (is0, is1)
        gsem = (gs0, gs1)
        osem = (os0, os1)

        def start_idx(j, s):
            pltpu.async_copy(idx_hbm.at[pl.ds(w0 + j * C, C)], idxs[s],
                             isem[s])

        def wait_idx(s):
            pltpu.make_async_copy(idx_hbm.at[pl.ds(w0, C)], idxs[s],
                                  isem[s]).wait()

        def start_gather(s):
            pltpu.async_copy(table_hbm.at[idxs[s]], rows[s], gsem[s])

        def wait_gather(s):
            pltpu.make_async_copy(out_hbm.at[pl.ds(w0, C)], rows[s],
                                  gsem[s]).wait()

        def start_out(j, s):
            pltpu.async_copy(rows[s], out_hbm.at[pl.ds(w0 + j * C, C)],
                             osem[s])

        def wait_out(s):
            pltpu.make_async_copy(rows[s], out_hbm.at[pl.ds(w0, C)],
                                  osem[s]).wait()

        start_idx(0, 0)
        start_idx(1, 1)
        wait_idx(0)
        start_gather(0)

        def process(j, s):
            @pl.when(j + 1 < n)
            def _():
                wait_idx(1 - s)

                @pl.when(j >= 1)
                def _():
                    wait_out(1 - s)

                start_gather(1 - s)

            wait_gather(s)
            start_out(j, s)

            @pl.when(j + 2 < n)
            def _():
                start_idx(j + 2, s)

        def body(g, carry):
            process(2 * g, 0)
            process(2 * g + 1, 1)
            return carry

        lax.fori_loop(0, n // 2, body, 0)
        wait_out((n - 2) % 2)
        wait_out((n - 1) % 2)

    return gather_k(table, idx)


def _sc_segment_sum(m_items, m_users, dst_idx, src_idx):
    C = 512
    n_iter = E_PAD // C
    mesh = plsc.VectorSubcoreMesh(core_axis_name="c", subcore_axis_name="s")

    @functools.partial(
        pl.kernel,
        mesh=mesh,
        out_type=(
            jax.ShapeDtypeStruct((F * N_PAD,), jnp.float32),
            jax.ShapeDtypeStruct((F * N_PAD,), jnp.float32),
        ),
        scratch_types=[
            pltpu.VMEM((C,), jnp.int32),
            pltpu.VMEM((C,), jnp.int32),
            pltpu.VMEM((16, C), jnp.float32),
            pltpu.VMEM((16, C), jnp.float32),
            pltpu.VMEM((C * 16,), jnp.float32),
            pltpu.VMEM((16 * N_PAD,), jnp.float32),
            pltpu.SemaphoreType.DMA,
            pltpu.SemaphoreType.DMA,
        ],
        compiler_params=pltpu.CompilerParams(needs_layout_passes=False),
    )
    def seg_k(mi_hbm, mu_hbm, di_hbm, si_hbm,
              item_out, user_out, idx0, idx1, rows0, rows1, flat_v, acc,
              sem0, sem1):
        cid = lax.axis_index("c")
        sid = lax.axis_index("s")
        col0 = sid * 16
        iot = lax.iota(jnp.int32, 16)
        iot16 = iot * 16
        iotN = iot * N_PAD
        zv = jnp.zeros((16,), jnp.float32)
        idxs = (idx0, idx1)
        rows = (rows0, rows1)
        sems = (sem0, sem1)

        def run(m_hbm, i_hbm, out_hbm):
            def start(j, s):
                base = j * C
                pltpu.async_copy(i_hbm.at[pl.ds(base, C)], idxs[s], sems[s])
                pltpu.async_copy(
                    m_hbm.at[pl.ds(col0, 16), pl.ds(base, C)], rows[s],
                    sems[s])

            def wait(s):
                pltpu.make_async_copy(i_hbm.at[pl.ds(0, C)], idxs[s],
                                      sems[s]).wait()
                pltpu.make_async_copy(
                    m_hbm.at[pl.ds(col0, 16), pl.ds(0, C)], rows[s],
                    sems[s]).wait()

            def zbody(r, carry):
                for u in range(8):
                    acc[pl.ds(r * 128 + u * 16, 16)] = zv
                return carry

            start(0, 0)
            start(1, 1)
            lax.fori_loop(0, N_PAD // 8, zbody, 0)

            def process(j, s):
                wait(s)

                def tbody(b, carry):
                    boff = jnp.full((16,), b * 256, jnp.int32)
                    for c in range(16):
                        v = rows[s][c, pl.ds(b * 16, 16)]
                        skew = iot16 + ((iot + c) & 15)
                        plsc.store_scatter(flat_v, [skew + boff], v)
                    return carry

                lax.fori_loop(0, C // 16, tbody, 0)

                @pl.when(j + 2 < n_iter)
                def _():
                    pltpu.async_copy(
                        m_hbm.at[pl.ds(col0, 16), pl.ds((j + 2) * C, C)],
                        rows[s], sems[s])

                def ebody(g, c2):
                    g256 = g * 256
                    dvec16 = idxs[s][pl.ds(g * 16, 16)] * 16
                    for l in range(16):
                        bv16 = jnp.take_along_axis(
                            dvec16, jnp.full((16,), l, jnp.int32), axis=0,
                            mode="promise_in_bounds")
                        v = flat_v[pl.ds(g256 + l * 16, 16)]
                        plsc.addupdate_scatter(
                            acc, [bv16 + ((iot - l) & 15)], v)
                    return c2

                lax.fori_loop(0, C // 16, ebody, 0)

                @pl.when(j + 2 < n_iter)
                def _():
                    pltpu.async_copy(i_hbm.at[pl.ds((j + 2) * C, C)],
                                     idxs[s], sems[s])

            def body(g, carry):
                process(2 * g, 0)
                process(2 * g + 1, 1)
                return carry

            lax.fori_loop(0, n_iter // 2, body, 0)
            pltpu.sync_copy(acc, out_hbm.at[pl.ds(col0 * N_PAD, 16 * N_PAD)])

        @pl.when(cid == 0)
        def _():
            run(mi_hbm, di_hbm, item_out)

        @pl.when(cid == 1)
        def _():
            run(mu_hbm, si_hbm, user_out)

    return seg_k(m_items, m_users, dst_idx, src_idx)


_BLK = 1024
_NBLK = E_PAD // _BLK


def _edge_dot_kernel(hs_ref, hd_ref, e_ref):
    i = pl.program_id(0)
    prod = hs_ref[...] * hd_ref[...]
    s = jnp.sum(prod.reshape(8, 128, F), axis=2) * (1.0 / 16.0)
    row = (i * _BLK
           + lax.broadcasted_iota(jnp.int32, (8, 128), 0) * 128
           + lax.broadcasted_iota(jnp.int32, (8, 128), 1))
    s = jnp.where(row < N_EDGES, s, -1e30)
    e_ref[...] = s.reshape(1, 8, 128)


def _tc_edge_dots(h_e):
    return pl.pallas_call(
        _edge_dot_kernel,
        grid=(_NBLK,),
        in_specs=[
            pl.BlockSpec((_BLK, F), lambda i: (i, 0)),
            pl.BlockSpec((_BLK, F), lambda i: (_NBLK + i, 0)),
        ],
        out_specs=pl.BlockSpec((1, 8, 128), lambda i: (i, 0, 0)),
        out_shape=jax.ShapeDtypeStruct((_NBLK, 8, 128), jnp.float32),
    )(h_e, h_e)


def _softmax_kernel(e_ref, a_ref):
    e = e_ref[...]
    m = jnp.max(e)
    ex = jnp.exp(e - m)
    a_ref[...] = ex * (1.0 / jnp.sum(ex))


def _tc_softmax(e3):
    return pl.pallas_call(
        _softmax_kernel,
        out_shape=jax.ShapeDtypeStruct((_NBLK, 8, 128), jnp.float32),
    )(e3)


def _msg_kernel(hs_ref, hd_ref, a_ref, ws_ref, bs_ref, wd_ref, bd_ref,
                mi_ref, mu_ref):
    alpha = a_ref[...].reshape(1, _BLK)
    fsT = jnp.maximum(
        lax.dot_general(ws_ref[...], hs_ref[...], (((1,), (1,)), ((), ())),
                        precision=lax.Precision.HIGHEST,
                        preferred_element_type=jnp.float32) + bs_ref[...], 0.0)
    mi_ref[...] = fsT * alpha
    fdT = jnp.maximum(
        lax.dot_general(wd_ref[...], hd_ref[...], (((1,), (1,)), ((), ())),
                        precision=lax.Precision.HIGHEST,
                        preferred_element_type=jnp.float32) + bd_ref[...], 0.0)
    mu_ref[...] = fdT * alpha


def _tc_messages(h_e, alpha3, Ws, bs2, Wd, bd2):
    return pl.pallas_call(
        _msg_kernel,
        grid=(_NBLK,),
        in_specs=[
            pl.BlockSpec((_BLK, F), lambda i: (i, 0)),
            pl.BlockSpec((_BLK, F), lambda i: (_NBLK + i, 0)),
            pl.BlockSpec((1, 8, 128), lambda i: (i, 0, 0)),
            pl.BlockSpec((F, F), lambda i: (0, 0)),
            pl.BlockSpec((F, 1), lambda i: (0, 0)),
            pl.BlockSpec((F, F), lambda i: (0, 0)),
            pl.BlockSpec((F, 1), lambda i: (0, 0)),
        ],
        out_specs=[
            pl.BlockSpec((F, _BLK), lambda i: (0, i)),
            pl.BlockSpec((F, _BLK), lambda i: (0, i)),
        ],
        out_shape=[
            jax.ShapeDtypeStruct((F, E_PAD), jnp.float32),
            jax.ShapeDtypeStruct((F, E_PAD), jnp.float32),
        ],
    )(h_e, h_e, alpha3, Ws, bs2, Wd, bd2)


def kernel(feat, user_ids, item_ids, edge_src, edge_dst,
           W_src, b_src, W_dst, b_dst):
    uid_p = jnp.pad(user_ids.astype(jnp.int32), (0, N_PAD - N_USERS))
    iid_p = jnp.pad(item_ids.astype(jnp.int32), (0, N_PAD - N_ITEMS))
    es_p = jnp.pad(edge_src.astype(jnp.int32), (0, E_PAD - N_EDGES))
    ed_p = jnp.pad(edge_dst.astype(jnp.int32), (0, E_PAD - N_EDGES))

    h = _sc_gather_rows(feat, jnp.concatenate([uid_p, iid_p]))
    h_e = _sc_gather_rows(h, jnp.concatenate([es_p, ed_p + N_PAD]))

    e3 = _tc_edge_dots(h_e)
    alpha3 = _tc_softmax(e3)

    m_items_T, m_users_T = _tc_messages(
        h_e, alpha3,
        W_src, b_src.reshape(F, 1),
        W_dst, b_dst.reshape(F, 1))

    item_1d, user_1d = _sc_segment_sum(m_items_T, m_users_T, ed_p, es_p)
    item_new = item_1d.reshape(16, N_PAD, 16).transpose(1, 0, 2).reshape(
        N_PAD, F)
    user_new = user_1d.reshape(16, N_PAD, 16).transpose(1, 0, 2).reshape(
        N_PAD, F)

    return jnp.concatenate(
        [user_new[:N_USERS], item_new[:N_ITEMS]], axis=0)


# speedup vs baseline: 1.9177x; 1.0376x over previous
import functools

import jax
import jax.numpy as jnp
from jax import lax
from jax.experimental import pallas as pl
from jax.experimental.pallas import tpu as pltpu
from jax.experimental.pallas import tpu_sc as plsc

F = 256
N_USERS = 5000
N_ITEMS = 5000
N_EDGES = 160000
NC, NS = 2, 16
NW = NC * NS
E_PAD = 163840
N_PAD = 5120


def _pick_chunk(n):
    for c in (128, 120, 112, 104, 96, 88, 80, 72, 64, 56, 48, 40, 32, 24, 16, 8):
        if n % c == 0:
            return c
    raise ValueError(n)


def _sc_gather_rows(table, idx):
    B = idx.shape[0]
    V, D = table.shape
    b_per_w = B // NW
    C = _pick_chunk(b_per_w)
    n = b_per_w // C
    assert n % 2 == 0
    mesh = plsc.VectorSubcoreMesh(core_axis_name="c", subcore_axis_name="s")

    @functools.partial(
        pl.kernel,
        mesh=mesh,
        out_type=jax.ShapeDtypeStruct((B, D), jnp.float32),
        scratch_types=[
            pltpu.VMEM((C,), jnp.int32),
            pltpu.VMEM((C,), jnp.int32),
            pltpu.VMEM((C, D), jnp.float32),
            pltpu.VMEM((C, D), jnp.float32),
            pltpu.SemaphoreType.DMA,
            pltpu.SemaphoreType.DMA,
            pltpu.SemaphoreType.DMA,
            pltpu.SemaphoreType.DMA,
            pltpu.SemaphoreType.DMA,
            pltpu.SemaphoreType.DMA,
        ],
    )
    def gather_k(table_hbm, idx_hbm, out_hbm, idx0, idx1, rows0, rows1,
                 is0, is1, gs0, gs1, os0, os1):
        wid = lax.axis_index("s") * NC + lax.axis_index("c")
        w0 = wid * b_per_w
        idxs = (idx0, idx1)
        rows = (rows0, rows1)
        isem = (is0, is1)
        gsem = (gs0, gs1)
        osem = (os0, os1)

        def start_idx(j, s):
            pltpu.async_copy(idx_hbm.at[pl.ds(w0 + j * C, C)], idxs[s],
                             isem[s])

        def wait_idx(s):
            pltpu.make_async_copy(idx_hbm.at[pl.ds(w0, C)], idxs[s],
                                  isem[s]).wait()

        def start_gather(s):
            pltpu.async_copy(table_hbm.at[idxs[s]], rows[s], gsem[s])

        def wait_gather(s):
            pltpu.make_async_copy(out_hbm.at[pl.ds(w0, C)], rows[s],
                                  gsem[s]).wait()

        def start_out(j, s):
            pltpu.async_copy(rows[s], out_hbm.at[pl.ds(w0 + j * C, C)],
                             osem[s])

        def wait_out(s):
            pltpu.make_async_copy(rows[s], out_hbm.at[pl.ds(w0, C)],
                                  osem[s]).wait()

        start_idx(0, 0)
        start_idx(1, 1)
        wait_idx(0)
        start_gather(0)

        def process(j, s):
            @pl.when(j + 1 < n)
            def _():
                wait_idx(1 - s)

                @pl.when(j >= 1)
                def _():
                    wait_out(1 - s)

                start_gather(1 - s)

            wait_gather(s)
            start_out(j, s)

            @pl.when(j + 2 < n)
            def _():
                start_idx(j + 2, s)

        def body(g, carry):
            process(2 * g, 0)
            process(2 * g + 1, 1)
            return carry

        lax.fori_loop(0, n // 2, body, 0)
        wait_out((n - 2) % 2)
        wait_out((n - 1) % 2)

    return gather_k(table, idx)


def _sc_segment_sum(m_items, m_users, dst_idx, src_idx):
    C = 512
    n_iter = E_PAD // C
    mesh = plsc.VectorSubcoreMesh(core_axis_name="c", subcore_axis_name="s")

    @functools.partial(
        pl.kernel,
        mesh=mesh,
        out_type=(
            jax.ShapeDtypeStruct((F * N_PAD,), jnp.float32),
            jax.ShapeDtypeStruct((F * N_PAD,), jnp.float32),
        ),
        scratch_types=[
            pltpu.VMEM((C,), jnp.int32),
            pltpu.VMEM((C,), jnp.int32),
            pltpu.VMEM((16, C), jnp.float32),
            pltpu.VMEM((16, C), jnp.float32),
            pltpu.VMEM((C * 16,), jnp.float32),
            pltpu.VMEM((16 * N_PAD,), jnp.float32),
            pltpu.SemaphoreType.DMA,
            pltpu.SemaphoreType.DMA,
        ],
        compiler_params=pltpu.CompilerParams(needs_layout_passes=False),
    )
    def seg_k(mi_hbm, mu_hbm, di_hbm, si_hbm,
              item_out, user_out, idx0, idx1, rows0, rows1, flat_v, acc,
              sem0, sem1):
        cid = lax.axis_index("c")
        sid = lax.axis_index("s")
        col0 = sid * 16
        iot = lax.iota(jnp.int32, 16)
        iot16 = iot * 16
        iotN = iot * N_PAD
        zv = jnp.zeros((16,), jnp.float32)
        idxs = (idx0, idx1)
        rows = (rows0, rows1)
        sems = (sem0, sem1)

        def run(m_hbm, i_hbm, out_hbm):
            def start(j, s):
                base = j * C
                pltpu.async_copy(i_hbm.at[pl.ds(base, C)], idxs[s], sems[s])
                pltpu.async_copy(
                    m_hbm.at[pl.ds(col0, 16), pl.ds(base, C)], rows[s],
                    sems[s])

            def wait(s):
                pltpu.make_async_copy(i_hbm.at[pl.ds(0, C)], idxs[s],
                                      sems[s]).wait()
                pltpu.make_async_copy(
                    m_hbm.at[pl.ds(col0, 16), pl.ds(0, C)], rows[s],
                    sems[s]).wait()

            def zbody(r, carry):
                for u in range(8):
                    acc[pl.ds(r * 128 + u * 16, 16)] = zv
                return carry

            start(0, 0)
            start(1, 1)
            lax.fori_loop(0, N_PAD // 8, zbody, 0)

            def process(j, s):
                wait(s)

                def tbody(b, carry):
                    boff = jnp.full((16,), b * 256, jnp.int32)
                    for c in range(16):
                        v = rows[s][c, pl.ds(b * 16, 16)]
                        skew = iot16 + ((iot + c) & 15)
                        plsc.store_scatter(flat_v, [skew + boff], v)
                    return carry

                lax.fori_loop(0, C // 16, tbody, 0)

                @pl.when(j + 2 < n_iter)
                def _():
                    pltpu.async_copy(
                        m_hbm.at[pl.ds(col0, 16), pl.ds((j + 2) * C, C)],
                        rows[s], sems[s])

                def ebody(g, c2):
                    g256 = g * 256
                    dvec16 = idxs[s][pl.ds(g * 16, 16)] * 16
                    for l in range(16):
                        bv16 = jnp.take_along_axis(
                            dvec16, jnp.full((16,), l, jnp.int32), axis=0,
                            mode="promise_in_bounds")
                        v = flat_v[pl.ds(g256 + l * 16, 16)]
                        plsc.addupdate_scatter(
                            acc, [bv16 + ((iot - l) & 15)], v)
                    return c2

                lax.fori_loop(0, C // 16, ebody, 0)

                @pl.when(j + 2 < n_iter)
                def _():
                    pltpu.async_copy(i_hbm.at[pl.ds((j + 2) * C, C)],
                                     idxs[s], sems[s])

            def body(g, carry):
                process(2 * g, 0)
                process(2 * g + 1, 1)
                return carry

            lax.fori_loop(0, n_iter // 2, body, 0)
            pltpu.sync_copy(acc, out_hbm.at[pl.ds(col0 * N_PAD, 16 * N_PAD)])

        @pl.when(cid == 0)
        def _():
            run(mi_hbm, di_hbm, item_out)

        @pl.when(cid == 1)
        def _():
            run(mu_hbm, si_hbm, user_out)

    return seg_k(m_items, m_users, dst_idx, src_idx)


_BLK = 1024
_NBLK = E_PAD // _BLK


def _edge_dot_kernel(hs_ref, hd_ref, e_ref):
    i = pl.program_id(0)
    prod = hs_ref[...] * hd_ref[...]
    s = jnp.sum(prod.reshape(8, 128, F), axis=2) * (1.0 / 16.0)
    row = (i * _BLK
           + lax.broadcasted_iota(jnp.int32, (8, 128), 0) * 128
           + lax.broadcasted_iota(jnp.int32, (8, 128), 1))
    s = jnp.where(row < N_EDGES, s, -1e30)
    e_ref[...] = s.reshape(1, 8, 128)


def _tc_edge_dots(h_e):
    return pl.pallas_call(
        _edge_dot_kernel,
        grid=(_NBLK,),
        in_specs=[
            pl.BlockSpec((_BLK, F), lambda i: (i, 0)),
            pl.BlockSpec((_BLK, F), lambda i: (_NBLK + i, 0)),
        ],
        out_specs=pl.BlockSpec((1, 8, 128), lambda i: (i, 0, 0)),
        out_shape=jax.ShapeDtypeStruct((_NBLK, 8, 128), jnp.float32),
    )(h_e, h_e)


def _softmax_kernel(e_ref, a_ref):
    e = e_ref[...]
    m = jnp.max(e)
    ex = jnp.exp(e - m)
    a_ref[...] = ex * (1.0 / jnp.sum(ex))


def _tc_softmax(e3):
    return pl.pallas_call(
        _softmax_kernel,
        out_shape=jax.ShapeDtypeStruct((_NBLK, 8, 128), jnp.float32),
    )(e3)


def _msg_kernel(hs_ref, hd_ref, a_ref, ws_ref, bs_ref, wd_ref, bd_ref,
                mi_ref, mu_ref):
    alpha = a_ref[...].reshape(1, _BLK)
    fsT = jnp.maximum(
        lax.dot_general(ws_ref[...], hs_ref[...], (((1,), (1,)), ((), ())),
                        precision=lax.Precision.DEFAULT,
                        preferred_element_type=jnp.float32) + bs_ref[...], 0.0)
    mi_ref[...] = fsT * alpha
    fdT = jnp.maximum(
        lax.dot_general(wd_ref[...], hd_ref[...], (((1,), (1,)), ((), ())),
                        precision=lax.Precision.DEFAULT,
                        preferred_element_type=jnp.float32) + bd_ref[...], 0.0)
    mu_ref[...] = fdT * alpha


def _tc_messages(h_e, alpha3, Ws, bs2, Wd, bd2):
    return pl.pallas_call(
        _msg_kernel,
        grid=(_NBLK,),
        in_specs=[
            pl.BlockSpec((_BLK, F), lambda i: (i, 0)),
            pl.BlockSpec((_BLK, F), lambda i: (_NBLK + i, 0)),
            pl.BlockSpec((1, 8, 128), lambda i: (i, 0, 0)),
            pl.BlockSpec((F, F), lambda i: (0, 0)),
            pl.BlockSpec((F, 1), lambda i: (0, 0)),
            pl.BlockSpec((F, F), lambda i: (0, 0)),
            pl.BlockSpec((F, 1), lambda i: (0, 0)),
        ],
        out_specs=[
            pl.BlockSpec((F, _BLK), lambda i: (0, i)),
            pl.BlockSpec((F, _BLK), lambda i: (0, i)),
        ],
        out_shape=[
            jax.ShapeDtypeStruct((F, E_PAD), jnp.float32),
            jax.ShapeDtypeStruct((F, E_PAD), jnp.float32),
        ],
    )(h_e, h_e, alpha3, Ws, bs2, Wd, bd2)


def kernel(feat, user_ids, item_ids, edge_src, edge_dst,
           W_src, b_src, W_dst, b_dst):
    uid_p = jnp.pad(user_ids.astype(jnp.int32), (0, N_PAD - N_USERS))
    iid_p = jnp.pad(item_ids.astype(jnp.int32), (0, N_PAD - N_ITEMS))
    es_p = jnp.pad(edge_src.astype(jnp.int32), (0, E_PAD - N_EDGES))
    ed_p = jnp.pad(edge_dst.astype(jnp.int32), (0, E_PAD - N_EDGES))

    h = _sc_gather_rows(feat, jnp.concatenate([uid_p, iid_p]))
    h_e = _sc_gather_rows(h, jnp.concatenate([es_p, ed_p + N_PAD]))

    e3 = _tc_edge_dots(h_e)
    alpha3 = _tc_softmax(e3)

    m_items_T, m_users_T = _tc_messages(
        h_e, alpha3,
        W_src, b_src.reshape(F, 1),
        W_dst, b_dst.reshape(F, 1))

    item_1d, user_1d = _sc_segment_sum(m_items_T, m_users_T, ed_p, es_p)
    item_new = item_1d.reshape(16, N_PAD, 16).transpose(1, 0, 2).reshape(
        N_PAD, F)
    user_new = user_1d.reshape(16, N_PAD, 16).transpose(1, 0, 2).reshape(
        N_PAD, F)

    return jnp.concatenate(
        [user_new[:N_USERS], item_new[:N_ITEMS]], axis=0)


# speedup vs baseline: 1.9769x; 1.0309x over previous
import functools

import jax
import jax.numpy as jnp
from jax import lax
from jax.experimental import pallas as pl
from jax.experimental.pallas import tpu as pltpu
from jax.experimental.pallas import tpu_sc as plsc

F = 256
N_USERS = 5000
N_ITEMS = 5000
N_EDGES = 160000
NC, NS = 2, 16
NW = NC * NS
E_PAD = 163840
N_PAD = 5120


def _pick_chunk(n):
    for c in (128, 120, 112, 104, 96, 88, 80, 72, 64, 56, 48, 40, 32, 24, 16, 8):
        if n % c == 0:
            return c
    raise ValueError(n)


def _sc_gather_rows(table, idx):
    B = idx.shape[0]
    V, D = table.shape
    b_per_w = B // NW
    C = _pick_chunk(b_per_w)
    n = b_per_w // C
    assert n % 2 == 0
    mesh = plsc.VectorSubcoreMesh(core_axis_name="c", subcore_axis_name="s")

    @functools.partial(
        pl.kernel,
        mesh=mesh,
        out_type=jax.ShapeDtypeStruct((B, D), table.dtype),
        scratch_types=[
            pltpu.VMEM((C,), jnp.int32),
            pltpu.VMEM((C,), jnp.int32),
            pltpu.VMEM((C, D), table.dtype),
            pltpu.VMEM((C, D), table.dtype),
            pltpu.SemaphoreType.DMA,
            pltpu.SemaphoreType.DMA,
            pltpu.SemaphoreType.DMA,
            pltpu.SemaphoreType.DMA,
            pltpu.SemaphoreType.DMA,
            pltpu.SemaphoreType.DMA,
        ],
    )
    def gather_k(table_hbm, idx_hbm, out_hbm, idx0, idx1, rows0, rows1,
                 is0, is1, gs0, gs1, os0, os1):
        wid = lax.axis_index("s") * NC + lax.axis_index("c")
        w0 = wid * b_per_w
        idxs = (idx0, idx1)
        rows = (rows0, rows1)
        isem = (is0, is1)
        gsem = (gs0, gs1)
        osem = (os0, os1)

        def start_idx(j, s):
            pltpu.async_copy(idx_hbm.at[pl.ds(w0 + j * C, C)], idxs[s],
                             isem[s])

        def wait_idx(s):
            pltpu.make_async_copy(idx_hbm.at[pl.ds(w0, C)], idxs[s],
                                  isem[s]).wait()

        def start_gather(s):
            pltpu.async_copy(table_hbm.at[idxs[s]], rows[s], gsem[s])

        def wait_gather(s):
            pltpu.make_async_copy(out_hbm.at[pl.ds(w0, C)], rows[s],
                                  gsem[s]).wait()

        def start_out(j, s):
            pltpu.async_copy(rows[s], out_hbm.at[pl.ds(w0 + j * C, C)],
                             osem[s])

        def wait_out(s):
            pltpu.make_async_copy(rows[s], out_hbm.at[pl.ds(w0, C)],
                                  osem[s]).wait()

        start_idx(0, 0)
        start_idx(1, 1)
        wait_idx(0)
        start_gather(0)

        def process(j, s):
            @pl.when(j + 1 < n)
            def _():
                wait_idx(1 - s)

                @pl.when(j >= 1)
                def _():
                    wait_out(1 - s)

                start_gather(1 - s)

            wait_gather(s)
            start_out(j, s)

            @pl.when(j + 2 < n)
            def _():
                start_idx(j + 2, s)

        def body(g, carry):
            process(2 * g, 0)
            process(2 * g + 1, 1)
            return carry

        lax.fori_loop(0, n // 2, body, 0)
        wait_out((n - 2) % 2)
        wait_out((n - 1) % 2)

    return gather_k(table, idx)


def _sc_segment_sum(m_items, m_users, dst_idx, src_idx):
    C = 512
    n_iter = E_PAD // C
    mesh = plsc.VectorSubcoreMesh(core_axis_name="c", subcore_axis_name="s")

    @functools.partial(
        pl.kernel,
        mesh=mesh,
        out_type=(
            jax.ShapeDtypeStruct((F * N_PAD,), jnp.float32),
            jax.ShapeDtypeStruct((F * N_PAD,), jnp.float32),
        ),
        scratch_types=[
            pltpu.VMEM((C,), jnp.int32),
            pltpu.VMEM((C,), jnp.int32),
            pltpu.VMEM((16, C), jnp.float32),
            pltpu.VMEM((16, C), jnp.float32),
            pltpu.VMEM((C * 16,), jnp.float32),
            pltpu.VMEM((16 * N_PAD,), jnp.float32),
            pltpu.SemaphoreType.DMA,
            pltpu.SemaphoreType.DMA,
        ],
        compiler_params=pltpu.CompilerParams(needs_layout_passes=False),
    )
    def seg_k(mi_hbm, mu_hbm, di_hbm, si_hbm,
              item_out, user_out, idx0, idx1, rows0, rows1, flat_v, acc,
              sem0, sem1):
        cid = lax.axis_index("c")
        sid = lax.axis_index("s")
        col0 = sid * 16
        iot = lax.iota(jnp.int32, 16)
        iot16 = iot * 16
        iotN = iot * N_PAD
        zv = jnp.zeros((16,), jnp.float32)
        idxs = (idx0, idx1)
        rows = (rows0, rows1)
        sems = (sem0, sem1)

        def run(m_hbm, i_hbm, out_hbm):
            def start(j, s):
                base = j * C
                pltpu.async_copy(i_hbm.at[pl.ds(base, C)], idxs[s], sems[s])
                pltpu.async_copy(
                    m_hbm.at[pl.ds(col0, 16), pl.ds(base, C)], rows[s],
                    sems[s])

            def wait(s):
                pltpu.make_async_copy(i_hbm.at[pl.ds(0, C)], idxs[s],
                                      sems[s]).wait()
                pltpu.make_async_copy(
                    m_hbm.at[pl.ds(col0, 16), pl.ds(0, C)], rows[s],
                    sems[s]).wait()

            def zbody(r, carry):
                for u in range(8):
                    acc[pl.ds(r * 128 + u * 16, 16)] = zv
                return carry

            start(0, 0)
            start(1, 1)
            lax.fori_loop(0, N_PAD // 8, zbody, 0)

            def process(j, s):
                wait(s)

                def tbody(b, carry):
                    boff = jnp.full((16,), b * 256, jnp.int32)
                    for c in range(16):
                        v = rows[s][c, pl.ds(b * 16, 16)]
                        skew = iot16 + ((iot + c) & 15)
                        plsc.store_scatter(flat_v, [skew + boff], v)
                    return carry

                lax.fori_loop(0, C // 16, tbody, 0)

                @pl.when(j + 2 < n_iter)
                def _():
                    pltpu.async_copy(
                        m_hbm.at[pl.ds(col0, 16), pl.ds((j + 2) * C, C)],
                        rows[s], sems[s])

                def ebody(g, c2):
                    g256 = g * 256
                    dvec16 = idxs[s][pl.ds(g * 16, 16)] * 16
                    for l in range(16):
                        bv16 = jnp.take_along_axis(
                            dvec16, jnp.full((16,), l, jnp.int32), axis=0,
                            mode="promise_in_bounds")
                        v = flat_v[pl.ds(g256 + l * 16, 16)]
                        plsc.addupdate_scatter(
                            acc, [bv16 + ((iot - l) & 15)], v)
                    return c2

                lax.fori_loop(0, C // 16, ebody, 0)

                @pl.when(j + 2 < n_iter)
                def _():
                    pltpu.async_copy(i_hbm.at[pl.ds((j + 2) * C, C)],
                                     idxs[s], sems[s])

            def body(g, carry):
                process(2 * g, 0)
                process(2 * g + 1, 1)
                return carry

            lax.fori_loop(0, n_iter // 2, body, 0)
            pltpu.sync_copy(acc, out_hbm.at[pl.ds(col0 * N_PAD, 16 * N_PAD)])

        @pl.when(cid == 0)
        def _():
            run(mi_hbm, di_hbm, item_out)

        @pl.when(cid == 1)
        def _():
            run(mu_hbm, si_hbm, user_out)

    return seg_k(m_items, m_users, dst_idx, src_idx)


_BLK = 1024
_NBLK = E_PAD // _BLK


def _unpack_f32(x):
    lo = lax.bitcast_convert_type(x << 16, jnp.float32)
    hi = lax.bitcast_convert_type(x & jnp.int32(-65536), jnp.float32)
    return lo, hi


def _edge_dot_kernel(hs_ref, hd_ref, e_ref):
    i = pl.program_id(0)
    slo, shi = _unpack_f32(hs_ref[...])
    dlo, dhi = _unpack_f32(hd_ref[...])
    prod = slo * dlo + shi * dhi
    s = jnp.sum(prod.reshape(8, 128, F // 2), axis=2) * (1.0 / 16.0)
    row = (i * _BLK
           + lax.broadcasted_iota(jnp.int32, (8, 128), 0) * 128
           + lax.broadcasted_iota(jnp.int32, (8, 128), 1))
    s = jnp.where(row < N_EDGES, s, -1e30)
    e_ref[...] = s.reshape(1, 8, 128)


def _tc_edge_dots(h_e):
    return pl.pallas_call(
        _edge_dot_kernel,
        grid=(_NBLK,),
        in_specs=[
            pl.BlockSpec((_BLK, F // 2), lambda i: (i, 0)),
            pl.BlockSpec((_BLK, F // 2), lambda i: (_NBLK + i, 0)),
        ],
        out_specs=pl.BlockSpec((1, 8, 128), lambda i: (i, 0, 0)),
        out_shape=jax.ShapeDtypeStruct((_NBLK, 8, 128), jnp.float32),
    )(h_e, h_e)


def _softmax_kernel(e_ref, a_ref):
    e = e_ref[...]
    m = jnp.max(e)
    ex = jnp.exp(e - m)
    a_ref[...] = ex * (1.0 / jnp.sum(ex))


def _tc_softmax(e3):
    return pl.pallas_call(
        _softmax_kernel,
        out_shape=jax.ShapeDtypeStruct((_NBLK, 8, 128), jnp.float32),
    )(e3)


def _msg_kernel(hs_ref, hd_ref, a_ref, wse_ref, wso_ref, bs_ref,
                wde_ref, wdo_ref, bd_ref, mi_ref, mu_ref):
    alpha = a_ref[...].reshape(1, _BLK)
    slo, shi = _unpack_f32(hs_ref[...])
    dlo, dhi = _unpack_f32(hd_ref[...])
    slo = slo.astype(jnp.bfloat16)
    shi = shi.astype(jnp.bfloat16)
    dlo = dlo.astype(jnp.bfloat16)
    dhi = dhi.astype(jnp.bfloat16)
    dn = (((1,), (1,)), ((), ()))

    def mm(w, x):
        return lax.dot_general(w, x, dn, precision=lax.Precision.DEFAULT,
                               preferred_element_type=jnp.float32)

    fsT = jnp.maximum(
        mm(wse_ref[...], slo) + mm(wso_ref[...], shi) + bs_ref[...], 0.0)
    mi_ref[...] = fsT * alpha
    fdT = jnp.maximum(
        mm(wde_ref[...], dlo) + mm(wdo_ref[...], dhi) + bd_ref[...], 0.0)
    mu_ref[...] = fdT * alpha


def _tc_messages(h_e, alpha3, Wse, Wso, bs2, Wde, Wdo, bd2):
    return pl.pallas_call(
        _msg_kernel,
        grid=(_NBLK,),
        in_specs=[
            pl.BlockSpec((_BLK, F // 2), lambda i: (i, 0)),
            pl.BlockSpec((_BLK, F // 2), lambda i: (_NBLK + i, 0)),
            pl.BlockSpec((1, 8, 128), lambda i: (i, 0, 0)),
            pl.BlockSpec((F, F // 2), lambda i: (0, 0)),
            pl.BlockSpec((F, F // 2), lambda i: (0, 0)),
            pl.BlockSpec((F, 1), lambda i: (0, 0)),
            pl.BlockSpec((F, F // 2), lambda i: (0, 0)),
            pl.BlockSpec((F, F // 2), lambda i: (0, 0)),
            pl.BlockSpec((F, 1), lambda i: (0, 0)),
        ],
        out_specs=[
            pl.BlockSpec((F, _BLK), lambda i: (0, i)),
            pl.BlockSpec((F, _BLK), lambda i: (0, i)),
        ],
        out_shape=[
            jax.ShapeDtypeStruct((F, E_PAD), jnp.float32),
            jax.ShapeDtypeStruct((F, E_PAD), jnp.float32),
        ],
    )(h_e, h_e, alpha3, Wse, Wso, bs2, Wde, Wdo, bd2)


def kernel(feat, user_ids, item_ids, edge_src, edge_dst,
           W_src, b_src, W_dst, b_dst):
    uid_p = jnp.pad(user_ids.astype(jnp.int32), (0, N_PAD - N_USERS))
    iid_p = jnp.pad(item_ids.astype(jnp.int32), (0, N_PAD - N_ITEMS))
    es_p = jnp.pad(edge_src.astype(jnp.int32), (0, E_PAD - N_EDGES))
    ed_p = jnp.pad(edge_dst.astype(jnp.int32), (0, E_PAD - N_EDGES))

    feat_p = lax.bitcast_convert_type(
        feat.astype(jnp.bfloat16).reshape(feat.shape[0], F // 2, 2),
        jnp.int32)
    h = _sc_gather_rows(feat_p, jnp.concatenate([uid_p, iid_p]))
    h_e = _sc_gather_rows(h, jnp.concatenate([es_p, ed_p + N_PAD]))

    e3 = _tc_edge_dots(h_e)
    alpha3 = _tc_softmax(e3)

    Wsb = W_src.astype(jnp.bfloat16)
    Wdb = W_dst.astype(jnp.bfloat16)
    m_items_T, m_users_T = _tc_messages(
        h_e, alpha3,
        Wsb[:, 0::2], Wsb[:, 1::2], b_src.reshape(F, 1),
        Wdb[:, 0::2], Wdb[:, 1::2], b_dst.reshape(F, 1))

    item_1d, user_1d = _sc_segment_sum(m_items_T, m_users_T, ed_p, es_p)
    item_new = item_1d.reshape(16, N_PAD, 16).transpose(1, 0, 2).reshape(
        N_PAD, F)
    user_new = user_1d.reshape(16, N_PAD, 16).transpose(1, 0, 2).reshape(
        N_PAD, F)

    return jnp.concatenate(
        [user_new[:N_USERS], item_new[:N_ITEMS]], axis=0)


# speedup vs baseline: 1.9792x; 1.0012x over previous
import functools

import jax
import jax.numpy as jnp
from jax import lax
from jax.experimental import pallas as pl
from jax.experimental.pallas import tpu as pltpu
from jax.experimental.pallas import tpu_sc as plsc

F = 256
N_USERS = 5000
N_ITEMS = 5000
N_EDGES = 160000
NC, NS = 2, 16
NW = NC * NS
E_PAD = 163840
N_PAD = 5120


def _pick_chunk(n):
    for c in (128, 120, 112, 104, 96, 88, 80, 72, 64, 56, 48, 40, 32, 24, 16, 8):
        if n % c == 0:
            return c
    raise ValueError(n)


def _sc_gather_rows(table, idx):
    B = idx.shape[0]
    V, D = table.shape
    b_per_w = B // NW
    C = _pick_chunk(b_per_w)
    n = b_per_w // C
    assert n % 2 == 0
    mesh = plsc.VectorSubcoreMesh(core_axis_name="c", subcore_axis_name="s")

    @functools.partial(
        pl.kernel,
        mesh=mesh,
        out_type=jax.ShapeDtypeStruct((B, D), table.dtype),
        scratch_types=[
            pltpu.VMEM((C,), jnp.int32),
            pltpu.VMEM((C,), jnp.int32),
            pltpu.VMEM((C, D), table.dtype),
            pltpu.VMEM((C, D), table.dtype),
            pltpu.SemaphoreType.DMA,
            pltpu.SemaphoreType.DMA,
            pltpu.SemaphoreType.DMA,
            pltpu.SemaphoreType.DMA,
            pltpu.SemaphoreType.DMA,
            pltpu.SemaphoreType.DMA,
        ],
    )
    def gather_k(table_hbm, idx_hbm, out_hbm, idx0, idx1, rows0, rows1,
                 is0, is1, gs0, gs1, os0, os1):
        wid = lax.axis_index("s") * NC + lax.axis_index("c")
        w0 = wid * b_per_w
        idxs = (idx0, idx1)
        rows = (rows0, rows1)
        isem = (is0, is1)
        gsem = (gs0, gs1)
        osem = (os0, os1)

        def start_idx(j, s):
            pltpu.async_copy(idx_hbm.at[pl.ds(w0 + j * C, C)], idxs[s],
                             isem[s])

        def wait_idx(s):
            pltpu.make_async_copy(idx_hbm.at[pl.ds(w0, C)], idxs[s],
                                  isem[s]).wait()

        def start_gather(s):
            pltpu.async_copy(table_hbm.at[idxs[s]], rows[s], gsem[s])

        def wait_gather(s):
            pltpu.make_async_copy(out_hbm.at[pl.ds(w0, C)], rows[s],
                                  gsem[s]).wait()

        def start_out(j, s):
            pltpu.async_copy(rows[s], out_hbm.at[pl.ds(w0 + j * C, C)],
                             osem[s])

        def wait_out(s):
            pltpu.make_async_copy(rows[s], out_hbm.at[pl.ds(w0, C)],
                                  osem[s]).wait()

        start_idx(0, 0)
        start_idx(1, 1)
        wait_idx(0)
        start_gather(0)

        def process(j, s):
            @pl.when(j + 1 < n)
            def _():
                wait_idx(1 - s)

                @pl.when(j >= 1)
                def _():
                    wait_out(1 - s)

                start_gather(1 - s)

            wait_gather(s)
            start_out(j, s)

            @pl.when(j + 2 < n)
            def _():
                start_idx(j + 2, s)

        def body(g, carry):
            process(2 * g, 0)
            process(2 * g + 1, 1)
            return carry

        lax.fori_loop(0, n // 2, body, 0)
        wait_out((n - 2) % 2)
        wait_out((n - 1) % 2)

    return gather_k(table, idx)


def _sc_segment_sum(m_items, m_users, dst_idx, src_idx):
    C = 512
    n_iter = E_PAD // C
    mesh = plsc.VectorSubcoreMesh(core_axis_name="c", subcore_axis_name="s")

    @functools.partial(
        pl.kernel,
        mesh=mesh,
        out_type=(
            jax.ShapeDtypeStruct((F * N_PAD,), jnp.float32),
            jax.ShapeDtypeStruct((F * N_PAD,), jnp.float32),
        ),
        scratch_types=[
            pltpu.VMEM((C,), jnp.int32),
            pltpu.VMEM((C,), jnp.int32),
            pltpu.VMEM((16, C), jnp.float32),
            pltpu.VMEM((16, C), jnp.float32),
            pltpu.VMEM((C * 16,), jnp.float32),
            pltpu.VMEM((16 * N_PAD,), jnp.float32),
            pltpu.SemaphoreType.DMA,
            pltpu.SemaphoreType.DMA,
        ],
        compiler_params=pltpu.CompilerParams(needs_layout_passes=False),
    )
    def seg_k(mi_hbm, mu_hbm, di_hbm, si_hbm,
              item_out, user_out, idx0, idx1, rows0, rows1, flat_v, acc,
              sem0, sem1):
        cid = lax.axis_index("c")
        sid = lax.axis_index("s")
        col0 = sid * 16
        iot = lax.iota(jnp.int32, 16)
        iot16 = iot * 16
        iotN = iot * N_PAD
        zv = jnp.zeros((16,), jnp.float32)
        idxs = (idx0, idx1)
        rows = (rows0, rows1)
        sems = (sem0, sem1)

        def run(m_hbm, i_hbm, out_hbm):
            def start(j, s):
                base = j * C
                pltpu.async_copy(i_hbm.at[pl.ds(base, C)], idxs[s], sems[s])
                pltpu.async_copy(
                    m_hbm.at[pl.ds(col0, 16), pl.ds(base, C)], rows[s],
                    sems[s])

            def wait(s):
                pltpu.make_async_copy(i_hbm.at[pl.ds(0, C)], idxs[s],
                                      sems[s]).wait()
                pltpu.make_async_copy(
                    m_hbm.at[pl.ds(col0, 16), pl.ds(0, C)], rows[s],
                    sems[s]).wait()

            def zbody(r, carry):
                for u in range(8):
                    acc[pl.ds(r * 128 + u * 16, 16)] = zv
                return carry

            start(0, 0)
            start(1, 1)
            lax.fori_loop(0, N_PAD // 8, zbody, 0)

            def process(j, s):
                wait(s)

                def tbody(b, carry):
                    boff = jnp.full((16,), b * 256, jnp.int32)
                    for c in range(16):
                        v = rows[s][c, pl.ds(b * 16, 16)]
                        skew = iot16 + ((iot + c) & 15)
                        plsc.store_scatter(flat_v, [skew + boff], v)
                    return carry

                lax.fori_loop(0, C // 16, tbody, 0, unroll=2)

                @pl.when(j + 2 < n_iter)
                def _():
                    pltpu.async_copy(
                        m_hbm.at[pl.ds(col0, 16), pl.ds((j + 2) * C, C)],
                        rows[s], sems[s])

                def ebody(g, c2):
                    g256 = g * 256
                    dvec16 = idxs[s][pl.ds(g * 16, 16)] * 16
                    for l in range(16):
                        bv16 = jnp.take_along_axis(
                            dvec16, jnp.full((16,), l, jnp.int32), axis=0,
                            mode="promise_in_bounds")
                        v = flat_v[pl.ds(g256 + l * 16, 16)]
                        plsc.addupdate_scatter(
                            acc, [bv16 + ((iot - l) & 15)], v)
                    return c2

                lax.fori_loop(0, C // 16, ebody, 0, unroll=2)

                @pl.when(j + 2 < n_iter)
                def _():
                    pltpu.async_copy(i_hbm.at[pl.ds((j + 2) * C, C)],
                                     idxs[s], sems[s])

            def body(g, carry):
                process(2 * g, 0)
                process(2 * g + 1, 1)
                return carry

            lax.fori_loop(0, n_iter // 2, body, 0)
            pltpu.sync_copy(acc, out_hbm.at[pl.ds(col0 * N_PAD, 16 * N_PAD)])

        @pl.when(cid == 0)
        def _():
            run(mi_hbm, di_hbm, item_out)

        @pl.when(cid == 1)
        def _():
            run(mu_hbm, si_hbm, user_out)

    return seg_k(m_items, m_users, dst_idx, src_idx)


_BLK = 1024
_NBLK = E_PAD // _BLK


def _unpack_f32(x):
    lo = lax.bitcast_convert_type(x << 16, jnp.float32)
    hi = lax.bitcast_convert_type(x & jnp.int32(-65536), jnp.float32)
    return lo, hi


def _edge_dot_kernel(hs_ref, hd_ref, e_ref):
    i = pl.program_id(0)
    slo, shi = _unpack_f32(hs_ref[...])
    dlo, dhi = _unpack_f32(hd_ref[...])
    prod = slo * dlo + shi * dhi
    s = jnp.sum(prod.reshape(8, 128, F // 2), axis=2) * (1.0 / 16.0)
    row = (i * _BLK
           + lax.broadcasted_iota(jnp.int32, (8, 128), 0) * 128
           + lax.broadcasted_iota(jnp.int32, (8, 128), 1))
    s = jnp.where(row < N_EDGES, s, -1e30)
    e_ref[...] = s.reshape(1, 8, 128)


def _tc_edge_dots(h_e):
    return pl.pallas_call(
        _edge_dot_kernel,
        grid=(_NBLK,),
        in_specs=[
            pl.BlockSpec((_BLK, F // 2), lambda i: (i, 0)),
            pl.BlockSpec((_BLK, F // 2), lambda i: (_NBLK + i, 0)),
        ],
        out_specs=pl.BlockSpec((1, 8, 128), lambda i: (i, 0, 0)),
        out_shape=jax.ShapeDtypeStruct((_NBLK, 8, 128), jnp.float32),
    )(h_e, h_e)


def _softmax_kernel(e_ref, a_ref):
    e = e_ref[...]
    m = jnp.max(e)
    ex = jnp.exp(e - m)
    a_ref[...] = ex * (1.0 / jnp.sum(ex))


def _tc_softmax(e3):
    return pl.pallas_call(
        _softmax_kernel,
        out_shape=jax.ShapeDtypeStruct((_NBLK, 8, 128), jnp.float32),
    )(e3)


def _msg_kernel(hs_ref, hd_ref, a_ref, wse_ref, wso_ref, bs_ref,
                wde_ref, wdo_ref, bd_ref, mi_ref, mu_ref):
    alpha = a_ref[...].reshape(1, _BLK)
    slo, shi = _unpack_f32(hs_ref[...])
    dlo, dhi = _unpack_f32(hd_ref[...])
    slo = slo.astype(jnp.bfloat16)
    shi = shi.astype(jnp.bfloat16)
    dlo = dlo.astype(jnp.bfloat16)
    dhi = dhi.astype(jnp.bfloat16)
    dn = (((1,), (1,)), ((), ()))

    def mm(w, x):
        return lax.dot_general(w, x, dn, precision=lax.Precision.DEFAULT,
                               preferred_element_type=jnp.float32)

    fsT = jnp.maximum(
        mm(wse_ref[...], slo) + mm(wso_ref[...], shi) + bs_ref[...], 0.0)
    mi_ref[...] = fsT * alpha
    fdT = jnp.maximum(
        mm(wde_ref[...], dlo) + mm(wdo_ref[...], dhi) + bd_ref[...], 0.0)
    mu_ref[...] = fdT * alpha


def _tc_messages(h_e, alpha3, Wse, Wso, bs2, Wde, Wdo, bd2):
    return pl.pallas_call(
        _msg_kernel,
        grid=(_NBLK,),
        in_specs=[
            pl.BlockSpec((_BLK, F // 2), lambda i: (i, 0)),
            pl.BlockSpec((_BLK, F // 2), lambda i: (_NBLK + i, 0)),
            pl.BlockSpec((1, 8, 128), lambda i: (i, 0, 0)),
            pl.BlockSpec((F, F // 2), lambda i: (0, 0)),
            pl.BlockSpec((F, F // 2), lambda i: (0, 0)),
            pl.BlockSpec((F, 1), lambda i: (0, 0)),
            pl.BlockSpec((F, F // 2), lambda i: (0, 0)),
            pl.BlockSpec((F, F // 2), lambda i: (0, 0)),
            pl.BlockSpec((F, 1), lambda i: (0, 0)),
        ],
        out_specs=[
            pl.BlockSpec((F, _BLK), lambda i: (0, i)),
            pl.BlockSpec((F, _BLK), lambda i: (0, i)),
        ],
        out_shape=[
            jax.ShapeDtypeStruct((F, E_PAD), jnp.float32),
            jax.ShapeDtypeStruct((F, E_PAD), jnp.float32),
        ],
    )(h_e, h_e, alpha3, Wse, Wso, bs2, Wde, Wdo, bd2)


def kernel(feat, user_ids, item_ids, edge_src, edge_dst,
           W_src, b_src, W_dst, b_dst):
    uid_p = jnp.pad(user_ids.astype(jnp.int32), (0, N_PAD - N_USERS))
    iid_p = jnp.pad(item_ids.astype(jnp.int32), (0, N_PAD - N_ITEMS))
    es_p = jnp.pad(edge_src.astype(jnp.int32), (0, E_PAD - N_EDGES))
    ed_p = jnp.pad(edge_dst.astype(jnp.int32), (0, E_PAD - N_EDGES))

    feat_p = lax.bitcast_convert_type(
        feat.astype(jnp.bfloat16).reshape(feat.shape[0], F // 2, 2),
        jnp.int32)
    h = _sc_gather_rows(feat_p, jnp.concatenate([uid_p, iid_p]))
    h_e = _sc_gather_rows(h, jnp.concatenate([es_p, ed_p + N_PAD]))

    e3 = _tc_edge_dots(h_e)
    alpha3 = _tc_softmax(e3)

    Wsb = W_src.astype(jnp.bfloat16)
    Wdb = W_dst.astype(jnp.bfloat16)
    m_items_T, m_users_T = _tc_messages(
        h_e, alpha3,
        Wsb[:, 0::2], Wsb[:, 1::2], b_src.reshape(F, 1),
        Wdb[:, 0::2], Wdb[:, 1::2], b_dst.reshape(F, 1))

    item_1d, user_1d = _sc_segment_sum(m_items_T, m_users_T, ed_p, es_p)
    item_new = item_1d.reshape(16, N_PAD, 16).transpose(1, 0, 2).reshape(
        N_PAD, F)
    user_new = user_1d.reshape(16, N_PAD, 16).transpose(1, 0, 2).reshape(
        N_PAD, F)

    return jnp.concatenate(
        [user_new[:N_USERS], item_new[:N_ITEMS]], axis=0)
